# Initial kernel scaffold; baseline (speedup 1.0000x reference)
import jax
import jax.numpy as jnp
from jax.experimental import pallas as pl


def kernel(xyz, points, W0, b0, gamma0, beta0, W1, b1, gamma1, beta1, W2, b2, gamma2, beta2):
    raise NotImplementedError("write your pallas kernel here")


# speedup vs baseline: 13.7558x; 13.7558x over previous
import jax
import jax.numpy as jnp
from jax.experimental import pallas as pl

NPOINT = 512
RADIUS = 0.4
NSAMPLE = 32
B = 4
N = 8192
RBLK = 128
NCH = 8


def _fps_kernel(far0_ref, xyz_ref, cent_ref, newx_ref):
    x = xyz_ref[:, 0, :]
    y = xyz_ref[:, 1, :]
    z = xyz_ref[:, 2, :]
    far = far0_ref[...]
    iota_n = jax.lax.broadcasted_iota(jnp.int32, (B, N), 1)
    iota_p = jax.lax.broadcasted_iota(jnp.int32, (B, NPOINT), 1)

    dist0 = jnp.full((B, N), 1e10, jnp.float32)
    cent0 = jnp.zeros((B, NPOINT), jnp.int32)
    cxs0 = jnp.zeros((B, NPOINT), jnp.float32)
    cys0 = jnp.zeros((B, NPOINT), jnp.float32)
    czs0 = jnp.zeros((B, NPOINT), jnp.float32)

    def body(i, st):
        dist, far, cent, cxs, cys, czs = st
        sel = iota_p == i
        cent = jnp.where(sel, far, cent)
        oh = (iota_n == far).astype(jnp.float32)
        cx = jnp.sum(x * oh, axis=1, keepdims=True)
        cy = jnp.sum(y * oh, axis=1, keepdims=True)
        cz = jnp.sum(z * oh, axis=1, keepdims=True)
        cxs = jnp.where(sel, cx, cxs)
        cys = jnp.where(sel, cy, cys)
        czs = jnp.where(sel, cz, czs)
        d = (x - cx) ** 2 + (y - cy) ** 2 + (z - cz) ** 2
        dist = jnp.minimum(dist, d)
        m = jnp.max(dist, axis=1, keepdims=True)
        cand = jnp.where(dist == m, iota_n, N)
        far = jnp.min(cand, axis=1, keepdims=True)
        return (dist, far, cent, cxs, cys, czs)

    _, _, cent, cxs, cys, czs = jax.lax.fori_loop(
        0, NPOINT, body, (dist0, far, cent0, cxs0, cys0, czs0))
    cent_ref[...] = cent
    newx_ref[:, 0, :] = cxs
    newx_ref[:, 1, :] = cys
    newx_ref[:, 2, :] = czs


def _ballq_kernel(xyz3_ref, newx_ref, feat_ref, out_ref):
    X3 = xyz3_ref[0]
    C = newx_ref[0]
    F = feat_ref[0]
    xn = jnp.sum(X3 * X3, axis=0, keepdims=True)
    cn = jnp.sum(C * C, axis=1, keepdims=True)
    D = -2.0 * jnp.dot(C, X3, preferred_element_type=jnp.float32) + cn + xn
    mask = jnp.logical_not(D > RADIUS * RADIUS)
    maskf = mask.astype(jnp.float32)

    pos = maskf
    sh = 1
    while sh < N:
        pos = pos + jnp.concatenate(
            [jnp.zeros((RBLK, sh), jnp.float32), pos[:, : N - sh]], axis=1)
        sh *= 2
    cnt = pos[:, N - 1 :]

    cpad = jnp.concatenate(
        [C, jnp.zeros((RBLK, NCH - 3), jnp.float32)], axis=1)
    g0 = None
    for s in range(NSAMPLE):
        ind = jnp.where(mask & (pos == float(s + 1)), 1.0, 0.0)
        g = jnp.dot(ind, F, preferred_element_type=jnp.float32)
        if s == 0:
            g0 = g
        else:
            g = jnp.where(cnt > float(s), g, g0)
        out_ref[0, :, s, :] = g - cpad


def _mlp_kernel(x_ref, w0_ref, b0_ref, g0_ref, be0_ref,
                w1_ref, b1_ref, g1_ref, be1_ref,
                w2_ref, b2_ref, g2_ref, be2_ref, out_ref):
    h = x_ref[...]
    for wr, br, gr, ber in ((w0_ref, b0_ref, g0_ref, be0_ref),
                            (w1_ref, b1_ref, g1_ref, be1_ref),
                            (w2_ref, b2_ref, g2_ref, be2_ref)):
        h = jnp.dot(wr[...], h, preferred_element_type=jnp.float32) + br[...]
        mean = jnp.mean(h, axis=1, keepdims=True)
        var = jnp.mean((h - mean) ** 2, axis=1, keepdims=True)
        h = (h - mean) / jnp.sqrt(var + 1e-5)
        h = gr[...] * h + ber[...]
        h = jnp.maximum(h, 0.0)
    m2 = B * NPOINT
    acc = h[:, 0:m2]
    for s in range(1, NSAMPLE):
        acc = jnp.maximum(acc, h[:, s * m2 : (s + 1) * m2])
    out_ref[...] = acc


def kernel(xyz, points, W0, b0, gamma0, beta0, W1, b1, gamma1, beta1,
           W2, b2, gamma2, beta2):
    far0 = jax.random.randint(
        jax.random.key(42), (B,), 0, N, dtype=jnp.int32).reshape(B, 1)

    cent, new_xyz = pl.pallas_call(
        _fps_kernel,
        out_shape=(
            jax.ShapeDtypeStruct((B, NPOINT), jnp.int32),
            jax.ShapeDtypeStruct((B, 3, NPOINT), jnp.float32),
        ),
    )(far0, xyz)

    xyz_t = jnp.transpose(xyz, (0, 2, 1))
    points_t = jnp.transpose(points, (0, 2, 1))
    iota_col = jnp.broadcast_to(
        jnp.arange(N, dtype=jnp.float32)[None, :, None], (B, N, 1))
    feat = jnp.concatenate(
        [xyz_t, points_t, iota_col, jnp.zeros((B, N, 1), jnp.float32)], axis=2)
    newx_t = jnp.transpose(new_xyz, (0, 2, 1))

    nblk = NPOINT // RBLK
    grouped = pl.pallas_call(
        _ballq_kernel,
        grid=(B, nblk),
        in_specs=[
            pl.BlockSpec((1, 3, N), lambda b, r: (b, 0, 0)),
            pl.BlockSpec((1, RBLK, 3), lambda b, r: (b, r, 0)),
            pl.BlockSpec((1, N, NCH), lambda b, r: (b, 0, 0)),
        ],
        out_specs=pl.BlockSpec((1, RBLK, NSAMPLE, NCH),
                               lambda b, r: (b, r, 0, 0)),
        out_shape=jax.ShapeDtypeStruct((B, NPOINT, NSAMPLE, NCH), jnp.float32),
    )(xyz, newx_t, feat)

    x_in = jnp.transpose(grouped[..., :6], (3, 2, 0, 1)).reshape(
        6, NSAMPLE * B * NPOINT)

    out = pl.pallas_call(
        _mlp_kernel,
        out_shape=jax.ShapeDtypeStruct((64, B * NPOINT), jnp.float32),
    )(x_in,
      W0, b0.reshape(-1, 1), gamma0.reshape(-1, 1), beta0.reshape(-1, 1),
      W1, b1.reshape(-1, 1), gamma1.reshape(-1, 1), beta1.reshape(-1, 1),
      W2, b2.reshape(-1, 1), gamma2.reshape(-1, 1), beta2.reshape(-1, 1))

    new_points = jnp.transpose(out.reshape(64, B, NPOINT), (1, 0, 2))
    del cent
    return (new_xyz, new_points)


# speedup vs baseline: 21.0785x; 1.5323x over previous
import jax
import jax.numpy as jnp
from jax.experimental import pallas as pl

NPOINT = 512
RADIUS = 0.4
NSAMPLE = 32
B = 4
N = 8192
RBLK = 128
NCH = 8


def _fps_kernel(far0_ref, xyz_ref, newx_ref):
    x = xyz_ref[:, 0, :]
    y = xyz_ref[:, 1, :]
    z = xyz_ref[:, 2, :]
    far = far0_ref[...]
    iota_n = jax.lax.broadcasted_iota(jnp.int32, (B, N), 1)
    iota_p = jax.lax.broadcasted_iota(jnp.int32, (B, NPOINT), 1)

    dist0 = jnp.full((B, N), 1e10, jnp.float32)
    cxs0 = jnp.zeros((B, NPOINT), jnp.float32)
    cys0 = jnp.zeros((B, NPOINT), jnp.float32)
    czs0 = jnp.zeros((B, NPOINT), jnp.float32)

    def body(i, st):
        dist, far, cxs, cys, czs = st
        sel = iota_p == i
        oh = (iota_n == far).astype(jnp.float32)
        cx = jnp.sum(x * oh, axis=1, keepdims=True)
        cy = jnp.sum(y * oh, axis=1, keepdims=True)
        cz = jnp.sum(z * oh, axis=1, keepdims=True)
        cxs = jnp.where(sel, cx, cxs)
        cys = jnp.where(sel, cy, cys)
        czs = jnp.where(sel, cz, czs)
        d = (x - cx) ** 2 + (y - cy) ** 2 + (z - cz) ** 2
        dist = jnp.minimum(dist, d)
        m = jnp.max(dist, axis=1, keepdims=True)
        cand = jnp.where(dist == m, iota_n, N)
        far = jnp.min(cand, axis=1, keepdims=True)
        return (dist, far, cxs, cys, czs)

    _, _, cxs, cys, czs = jax.lax.fori_loop(
        0, NPOINT, body, (dist0, far, cxs0, cys0, czs0))
    newx_ref[:, 0, :] = cxs
    newx_ref[:, 1, :] = cys
    newx_ref[:, 2, :] = czs


def _ballq_kernel(xyzt_ref, xyz3_ref, pts3_ref, newx_ref, out_ref):
    X = xyzt_ref[0]
    CT = newx_ref[0]
    xn = jnp.sum(X * X, axis=1, keepdims=True)
    cn = jnp.sum(CT * CT, axis=0, keepdims=True)
    D = (-2.0 * jnp.dot(X, CT, preferred_element_type=jnp.float32)
         + cn) + xn
    mask = jnp.logical_not(D > RADIUS * RADIUS)
    maskf = mask.astype(jnp.float32)

    pos = maskf
    sh = 1
    while sh < N:
        pos = pos + jnp.concatenate(
            [jnp.zeros((sh, RBLK), jnp.float32), pos[: N - sh, :]], axis=0)
        sh *= 2
    cnt = pos[N - 1 :, :]
    V = jnp.where(mask, pos, 0.0)

    iota_row = jax.lax.broadcasted_iota(
        jnp.int32, (1, N), 1).astype(jnp.float32)
    FT = jnp.concatenate(
        [xyz3_ref[0], pts3_ref[0], iota_row,
         jnp.zeros((1, N), jnp.float32)], axis=0)
    cpad = jnp.concatenate(
        [CT, jnp.zeros((NCH - 3, RBLK), jnp.float32)], axis=0)

    g0 = None
    for s in range(NSAMPLE):
        ind = jnp.where(V == float(s + 1), 1.0, 0.0)
        g = jnp.dot(FT, ind, preferred_element_type=jnp.float32)
        if s == 0:
            g0 = g
        else:
            g = jnp.where(cnt > float(s), g, g0)
        out_ref[:, s, :] = g - cpad


def _mlp_kernel(x_ref, w0_ref, b0_ref, g0_ref, be0_ref,
                w1_ref, b1_ref, g1_ref, be1_ref,
                w2_ref, b2_ref, g2_ref, be2_ref, out_ref):
    h = x_ref[0:6, :]
    for wr, br, gr, ber in ((w0_ref, b0_ref, g0_ref, be0_ref),
                            (w1_ref, b1_ref, g1_ref, be1_ref),
                            (w2_ref, b2_ref, g2_ref, be2_ref)):
        h = jnp.dot(wr[...], h, preferred_element_type=jnp.float32) + br[...]
        mean = jnp.mean(h, axis=1, keepdims=True)
        var = jnp.mean((h - mean) ** 2, axis=1, keepdims=True)
        h = (h - mean) / jnp.sqrt(var + 1e-5)
        h = gr[...] * h + ber[...]
        h = jnp.maximum(h, 0.0)
    m2 = B * NPOINT
    acc = h[:, 0:m2]
    for s in range(1, NSAMPLE):
        acc = jnp.maximum(acc, h[:, s * m2 : (s + 1) * m2])
    for b in range(B):
        out_ref[b, :, :] = acc[:, b * NPOINT : (b + 1) * NPOINT]


def kernel(xyz, points, W0, b0, gamma0, beta0, W1, b1, gamma1, beta1,
           W2, b2, gamma2, beta2):
    far0 = jax.random.randint(
        jax.random.key(42), (B,), 0, N, dtype=jnp.int32).reshape(B, 1)

    new_xyz = pl.pallas_call(
        _fps_kernel,
        out_shape=jax.ShapeDtypeStruct((B, 3, NPOINT), jnp.float32),
    )(far0, xyz)

    xyz_t = jnp.transpose(xyz, (0, 2, 1))
    nblk = NPOINT // RBLK
    grouped = pl.pallas_call(
        _ballq_kernel,
        grid=(B, nblk),
        in_specs=[
            pl.BlockSpec((1, N, 3), lambda b, r: (b, 0, 0)),
            pl.BlockSpec((1, 3, N), lambda b, r: (b, 0, 0)),
            pl.BlockSpec((1, 3, N), lambda b, r: (b, 0, 0)),
            pl.BlockSpec((1, 3, RBLK), lambda b, r: (b, 0, r)),
        ],
        out_specs=pl.BlockSpec((NCH, NSAMPLE, RBLK),
                               lambda b, r: (0, 0, b * (NPOINT // RBLK) + r)),
        out_shape=jax.ShapeDtypeStruct((NCH, NSAMPLE, B * NPOINT),
                                       jnp.float32),
    )(xyz_t, xyz, points, new_xyz)

    x_in = grouped.reshape(NCH, NSAMPLE * B * NPOINT)

    new_points = pl.pallas_call(
        _mlp_kernel,
        out_shape=jax.ShapeDtypeStruct((B, 64, NPOINT), jnp.float32),
    )(x_in,
      W0, b0.reshape(-1, 1), gamma0.reshape(-1, 1), beta0.reshape(-1, 1),
      W1, b1.reshape(-1, 1), gamma1.reshape(-1, 1), beta1.reshape(-1, 1),
      W2, b2.reshape(-1, 1), gamma2.reshape(-1, 1), beta2.reshape(-1, 1))

    return (new_xyz, new_points)
